# Initial kernel scaffold; baseline (speedup 1.0000x reference)
import jax
import jax.numpy as jnp
from jax.experimental import pallas as pl


def kernel(s, segment_ids, W1, b1, W2, b2, W3, b3):
    raise NotImplementedError("write your pallas kernel here")


# speedup vs baseline: 2.8640x; 2.8640x over previous
import functools

import jax
import jax.numpy as jnp
from jax import lax
from jax.experimental import pallas as pl
from jax.experimental.pallas import tpu as pltpu
from jax.experimental.pallas import tpu_sc as plsc

N_NODES = 100000
NSEG = 512
D = 128
NC = 2
NS = 16
NW = NC * NS

C = 256
FULL_CHUNKS = N_NODES // C
TAIL = N_NODES - FULL_CHUNKS * C
N_CHUNKS = FULL_CHUNKS + 1
CROWS = NSEG // 8 + 1


def _sc_segment_sums(s, ids2d):
    mesh = plsc.VectorSubcoreMesh(core_axis_name="c", subcore_axis_name="s")

    @functools.partial(
        pl.kernel,
        mesh=mesh,
        out_type=(
            jax.ShapeDtypeStruct((NW, NSEG, D), jnp.float32),
            jax.ShapeDtypeStruct((NW, CROWS, D), jnp.float32),
        ),
        scratch_types=[
            pltpu.VMEM((NSEG + 1, D), jnp.float32),
            pltpu.VMEM((CROWS, D), jnp.float32),
            pltpu.VMEM((C, D), jnp.float32),
            pltpu.VMEM((C,), jnp.int32),
            pltpu.SemaphoreType.DMA,
            pltpu.SemaphoreType.DMA,
        ],
    )
    def seg_sum(s_hbm, ids_hbm, out_parts, cnt_parts, acc, cacc, buf, idxb,
                sem1, sem2):
        cid = lax.axis_index("c")
        sid = lax.axis_index("s")
        wid = sid * NC + cid

        zeros16 = jnp.zeros((16,), jnp.float32)
        ones16 = jnp.ones((16,), jnp.float32)

        def zero_acc(k, _):
            i = k // (D // 16)
            j = k % (D // 16)
            acc[i, pl.ds(j * 16, 16)] = zeros16
            return 0

        lax.fori_loop(0, (NSEG + 1) * (D // 16), zero_acc, 0)

        def zero_cacc(k, _):
            i = k // (D // 16)
            j = k % (D // 16)
            cacc[i, pl.ds(j * 16, 16)] = zeros16
            return 0

        lax.fori_loop(0, CROWS * (D // 16), zero_cacc, 0)

        n_my = (N_CHUNKS - 1 - wid) // NW + 1

        def body(t, _):
            chunk = t * NW + wid
            base = chunk * C
            cp_ids = pltpu.async_copy(ids_hbm.at[chunk], idxb, sem2)

            @pl.when(chunk < FULL_CHUNKS)
            def _():
                pltpu.async_copy(s_hbm.at[pl.ds(base, C)], buf, sem1).wait()

            @pl.when(chunk == FULL_CHUNKS)
            def _():
                pltpu.async_copy(
                    s_hbm.at[pl.ds(base, TAIL)], buf.at[pl.ds(0, TAIL)], sem1
                ).wait()

            cp_ids.wait()

            def group(g, _):
                rbase = g * 16
                ids16 = idxb[pl.ds(rbase, 16)]
                for r in range(16):
                    seg = ids16[r]
                    for j in range(D // 16):
                        x = buf[rbase + r, pl.ds(j * 16, 16)]
                        plsc.addupdate(acc.at[seg, pl.ds(j * 16, 16)], x)
                    plsc.addupdate(
                        cacc.at[seg // 8, pl.ds((seg % 8) * 16, 16)], ones16)
                return 0

            lax.fori_loop(0, C // 16, group, 0)
            return 0

        lax.fori_loop(0, n_my, body, 0)

        pltpu.sync_copy(acc.at[pl.ds(0, NSEG)], out_parts.at[wid])
        pltpu.sync_copy(cacc, cnt_parts.at[wid])

    return seg_sum(s, ids2d)


def _finalize_body(parts_ref, cnt_ref, w1_ref, b1_ref, w2_ref, b2_ref,
                   w3_ref, b3_ref, out_ref):
    sums = jnp.sum(parts_ref[...], axis=0)
    cnt = jnp.sum(cnt_ref[...], axis=0)
    srow = lax.broadcasted_iota(jnp.int32, (NSEG, CROWS), 0)
    crow = lax.broadcasted_iota(jnp.int32, (NSEG, CROWS), 1)
    sel = (crow == srow // 8).astype(jnp.float32)
    cntrows = jnp.dot(sel, cnt, preferred_element_type=jnp.float32)
    ss = lax.broadcasted_iota(jnp.int32, (NSEG, D), 0)
    kk = lax.broadcasted_iota(jnp.int32, (NSEG, D), 1)
    lane_mask = (kk == (ss % 8) * 16).astype(jnp.float32)
    counts = jnp.sum(cntrows * lane_mask, axis=1, keepdims=True)
    mean = sums / jnp.maximum(counts, 1.0)
    h = jnp.maximum(
        jnp.dot(mean, w1_ref[...], preferred_element_type=jnp.float32)
        + b1_ref[...], 0.0)
    h = jnp.maximum(
        jnp.dot(h, w2_ref[...], preferred_element_type=jnp.float32)
        + b2_ref[...], 0.0)
    out = jnp.sum(h * w3_ref[...], axis=1, keepdims=True) + b3_ref[...]
    out_ref[...] = out


def _finalize(parts, cnts, W1, b1, W2, b2, W3, b3):
    return pl.pallas_call(
        _finalize_body,
        out_shape=jax.ShapeDtypeStruct((NSEG, 1), jnp.float32),
    )(parts, cnts, W1, b1.reshape(1, -1), W2, b2.reshape(1, -1),
      W3.reshape(1, -1), b3.reshape(1, 1))


def kernel(s, segment_ids, W1, b1, W2, b2, W3, b3):
    ids = segment_ids.astype(jnp.int32)
    pad = jnp.full((N_CHUNKS * C - N_NODES,), NSEG, jnp.int32)
    ids2d = jnp.concatenate([ids, pad]).reshape(N_CHUNKS, C)
    parts, cnts = _sc_segment_sums(s, ids2d)
    return _finalize(parts, cnts, W1, b1, W2, b2, W3, b3)


# speedup vs baseline: 4.4940x; 1.5691x over previous
import functools

import jax
import jax.numpy as jnp
from jax import lax
from jax.experimental import pallas as pl
from jax.experimental.pallas import tpu as pltpu
from jax.experimental.pallas import tpu_sc as plsc

N_NODES = 100000
NSEG = 512
D = 128
NC = 2
NS = 16
NW = NC * NS

C = 256
FULL_CHUNKS = N_NODES // C
TAIL = N_NODES - FULL_CHUNKS * C
N_CHUNKS = FULL_CHUNKS + 1
CROWS = NSEG // 8 + 1


def _sc_segment_sums(s, ids2d):
    mesh = plsc.VectorSubcoreMesh(core_axis_name="c", subcore_axis_name="s")

    @functools.partial(
        pl.kernel,
        mesh=mesh,
        out_type=(
            jax.ShapeDtypeStruct((NW, NSEG, D), jnp.float32),
            jax.ShapeDtypeStruct((NW, CROWS, D), jnp.float32),
        ),
        scratch_types=[
            pltpu.VMEM((NSEG + 1, D), jnp.float32),
            pltpu.VMEM((CROWS, D), jnp.float32),
            pltpu.VMEM((C, D), jnp.float32),
            pltpu.VMEM((C,), jnp.int32),
            pltpu.SemaphoreType.DMA,
            pltpu.SemaphoreType.DMA,
        ],
    )
    def seg_sum(s_hbm, ids_hbm, out_parts, cnt_parts, acc, cacc, buf, idxb,
                sem1, sem2):
        cid = lax.axis_index("c")
        sid = lax.axis_index("s")
        wid = sid * NC + cid

        zeros16 = jnp.zeros((16,), jnp.float32)
        ones16 = jnp.ones((16,), jnp.float32)

        def zero_acc(k, _):
            i = k // (D // 16)
            j = k % (D // 16)
            acc[i, pl.ds(j * 16, 16)] = zeros16
            return 0

        lax.fori_loop(0, (NSEG + 1) * (D // 16), zero_acc, 0)

        def zero_cacc(k, _):
            i = k // (D // 16)
            j = k % (D // 16)
            cacc[i, pl.ds(j * 16, 16)] = zeros16
            return 0

        lax.fori_loop(0, CROWS * (D // 16), zero_cacc, 0)

        n_my = (N_CHUNKS - 1 - wid) // NW + 1

        def body(t, _):
            chunk = t * NW + wid
            base = chunk * C
            cp_ids = pltpu.async_copy(ids_hbm.at[chunk], idxb, sem2)

            @pl.when(chunk < FULL_CHUNKS)
            def _():
                pltpu.async_copy(s_hbm.at[pl.ds(base, C)], buf, sem1).wait()

            @pl.when(chunk == FULL_CHUNKS)
            def _():
                pltpu.async_copy(
                    s_hbm.at[pl.ds(base, TAIL)], buf.at[pl.ds(0, TAIL)], sem1
                ).wait()

            cp_ids.wait()

            def group(g, _):
                rbase = g * 16
                ids16 = idxb[pl.ds(rbase, 16)]

                def load_row(r):
                    return [buf[rbase + r, pl.ds(j * 16, 16)]
                            for j in range(D // 16)]

                def store_row(seg, xs):
                    for j in range(D // 16):
                        plsc.addupdate(acc.at[seg, pl.ds(j * 16, 16)], xs[j])
                    plsc.addupdate(
                        cacc.at[seg // 8, pl.ds((seg % 8) * 16, 16)], ones16)

                xs = load_row(0)
                seg = ids16[0]
                for r in range(16):
                    if r < 15:
                        nxt_xs = load_row(r + 1)
                        nxt_seg = ids16[r + 1]
                    store_row(seg, xs)
                    if r < 15:
                        xs, seg = nxt_xs, nxt_seg
                return 0

            lax.fori_loop(0, C // 16, group, 0)
            return 0

        lax.fori_loop(0, n_my, body, 0)

        pltpu.sync_copy(acc.at[pl.ds(0, NSEG)], out_parts.at[wid])
        pltpu.sync_copy(cacc, cnt_parts.at[wid])

    return seg_sum(s, ids2d)


def _finalize_body(parts_ref, cnt_ref, w1_ref, b1_ref, w2_ref, b2_ref,
                   w3_ref, b3_ref, out_ref):
    sums = jnp.sum(parts_ref[...], axis=0)
    cnt = jnp.sum(cnt_ref[...], axis=0)
    srow = lax.broadcasted_iota(jnp.int32, (NSEG, CROWS), 0)
    crow = lax.broadcasted_iota(jnp.int32, (NSEG, CROWS), 1)
    sel = (crow == srow // 8).astype(jnp.float32)
    cntrows = jnp.dot(sel, cnt, preferred_element_type=jnp.float32)
    ss = lax.broadcasted_iota(jnp.int32, (NSEG, D), 0)
    kk = lax.broadcasted_iota(jnp.int32, (NSEG, D), 1)
    lane_mask = (kk == (ss % 8) * 16).astype(jnp.float32)
    counts = jnp.sum(cntrows * lane_mask, axis=1, keepdims=True)
    mean = sums / jnp.maximum(counts, 1.0)
    h = jnp.maximum(
        jnp.dot(mean, w1_ref[...], preferred_element_type=jnp.float32)
        + b1_ref[...], 0.0)
    h = jnp.maximum(
        jnp.dot(h, w2_ref[...], preferred_element_type=jnp.float32)
        + b2_ref[...], 0.0)
    out = jnp.sum(h * w3_ref[...], axis=1, keepdims=True) + b3_ref[...]
    out_ref[...] = out


def _finalize(parts, cnts, W1, b1, W2, b2, W3, b3):
    return pl.pallas_call(
        _finalize_body,
        out_shape=jax.ShapeDtypeStruct((NSEG, 1), jnp.float32),
    )(parts, cnts, W1, b1.reshape(1, -1), W2, b2.reshape(1, -1),
      W3.reshape(1, -1), b3.reshape(1, 1))


def kernel(s, segment_ids, W1, b1, W2, b2, W3, b3):
    ids = segment_ids.astype(jnp.int32)
    pad = jnp.full((N_CHUNKS * C - N_NODES,), NSEG, jnp.int32)
    ids2d = jnp.concatenate([ids, pad]).reshape(N_CHUNKS, C)
    parts, cnts = _sc_segment_sums(s, ids2d)
    return _finalize(parts, cnts, W1, b1, W2, b2, W3, b3)


# speedup vs baseline: 5.0069x; 1.1141x over previous
import functools

import jax
import jax.numpy as jnp
from jax import lax
from jax.experimental import pallas as pl
from jax.experimental.pallas import tpu as pltpu
from jax.experimental.pallas import tpu_sc as plsc

N_NODES = 100000
NSEG = 512
D = 128
NC = 2
NS = 16
NW = NC * NS

C = 256
FULL_CHUNKS = N_NODES // C
TAIL = N_NODES - FULL_CHUNKS * C
N_CHUNKS = FULL_CHUNKS + 1
CROWS = NSEG // 8 + 1


def _sc_segment_sums(s, ids2d):
    mesh = plsc.VectorSubcoreMesh(core_axis_name="c", subcore_axis_name="s")

    @functools.partial(
        pl.kernel,
        mesh=mesh,
        out_type=(
            jax.ShapeDtypeStruct((NW, NSEG, D), jnp.float32),
            jax.ShapeDtypeStruct((NW, CROWS, D), jnp.float32),
        ),
        scratch_types=[
            pltpu.VMEM((NSEG + 1, D), jnp.float32),
            pltpu.VMEM((CROWS, D), jnp.float32),
            pltpu.VMEM((C, D), jnp.float32),
            pltpu.VMEM((C,), jnp.int32),
            pltpu.SemaphoreType.DMA,
            pltpu.SemaphoreType.DMA,
        ],
    )
    def seg_sum(s_hbm, ids_hbm, out_parts, cnt_parts, acc, cacc, buf, idxb,
                sem1, sem2):
        cid = lax.axis_index("c")
        sid = lax.axis_index("s")
        wid = sid * NC + cid

        zeros16 = jnp.zeros((16,), jnp.float32)
        ones16 = jnp.ones((16,), jnp.float32)
        sixteen16 = jnp.full((16,), 16.0, jnp.float32)

        def zero_acc(k, _):
            i = k // (D // 16)
            j = k % (D // 16)
            acc[i, pl.ds(j * 16, 16)] = zeros16
            return 0

        lax.fori_loop(0, (NSEG + 1) * (D // 16), zero_acc, 0)

        def zero_cacc(k, _):
            i = k // (D // 16)
            j = k % (D // 16)
            cacc[i, pl.ds(j * 16, 16)] = zeros16
            return 0

        lax.fori_loop(0, CROWS * (D // 16), zero_cacc, 0)

        n_my = (N_CHUNKS - 1 - wid) // NW + 1

        def body(t, _):
            chunk = t * NW + wid
            base = chunk * C
            cp_ids = pltpu.async_copy(ids_hbm.at[chunk], idxb, sem2)

            @pl.when(chunk < FULL_CHUNKS)
            def _():
                pltpu.async_copy(s_hbm.at[pl.ds(base, C)], buf, sem1).wait()

            @pl.when(chunk == FULL_CHUNKS)
            def _():
                pltpu.async_copy(
                    s_hbm.at[pl.ds(base, TAIL)], buf.at[pl.ds(0, TAIL)], sem1
                ).wait()

            cp_ids.wait()

            def group(g, _):
                rbase = g * 16
                ids16 = idxb[pl.ds(rbase, 16)]
                seg0 = ids16[0]
                seg15 = ids16[15]

                def load_row(r):
                    return [buf[rbase + r, pl.ds(j * 16, 16)]
                            for j in range(D // 16)]

                def store_row(seg, xs):
                    for j in range(D // 16):
                        plsc.addupdate(acc.at[seg, pl.ds(j * 16, 16)], xs[j])
                    plsc.addupdate(
                        cacc.at[seg // 8, pl.ds((seg % 8) * 16, 16)], ones16)

                @pl.when(seg0 == seg15)
                def _():
                    sums = load_row(0)
                    for r in range(1, 16):
                        xs = load_row(r)
                        for j in range(D // 16):
                            sums[j] = sums[j] + xs[j]
                    for j in range(D // 16):
                        plsc.addupdate(
                            acc.at[seg0, pl.ds(j * 16, 16)], sums[j])
                    plsc.addupdate(
                        cacc.at[seg0 // 8, pl.ds((seg0 % 8) * 16, 16)],
                        sixteen16)

                @pl.when(seg0 != seg15)
                def _():
                    xs = load_row(0)
                    seg = seg0
                    for r in range(16):
                        if r < 15:
                            nxt_seg = ids16[r + 1]
                            nxt_xs = []
                            for j in range(D // 16):
                                nxt_xs.append(buf[rbase + r + 1,
                                                  pl.ds(j * 16, 16)])
                                plsc.addupdate(
                                    acc.at[seg, pl.ds(j * 16, 16)], xs[j])
                            plsc.addupdate(
                                cacc.at[seg // 8, pl.ds((seg % 8) * 16, 16)],
                                ones16)
                            xs, seg = nxt_xs, nxt_seg
                        else:
                            store_row(seg, xs)
                return 0

            lax.fori_loop(0, C // 16, group, 0)
            return 0

        lax.fori_loop(0, n_my, body, 0)

        pltpu.sync_copy(acc.at[pl.ds(0, NSEG)], out_parts.at[wid])
        pltpu.sync_copy(cacc, cnt_parts.at[wid])

    return seg_sum(s, ids2d)


def _finalize_body(parts_ref, cnt_ref, w1_ref, b1_ref, w2_ref, b2_ref,
                   w3_ref, b3_ref, out_ref):
    sums = jnp.sum(parts_ref[...], axis=0)
    cnt = jnp.sum(cnt_ref[...], axis=0)
    srow = lax.broadcasted_iota(jnp.int32, (NSEG, CROWS), 0)
    crow = lax.broadcasted_iota(jnp.int32, (NSEG, CROWS), 1)
    sel = (crow == srow // 8).astype(jnp.float32)
    cntrows = jnp.dot(sel, cnt, preferred_element_type=jnp.float32)
    ss = lax.broadcasted_iota(jnp.int32, (NSEG, D), 0)
    kk = lax.broadcasted_iota(jnp.int32, (NSEG, D), 1)
    lane_mask = (kk == (ss % 8) * 16).astype(jnp.float32)
    counts = jnp.sum(cntrows * lane_mask, axis=1, keepdims=True)
    mean = sums / jnp.maximum(counts, 1.0)
    h = jnp.maximum(
        jnp.dot(mean, w1_ref[...], preferred_element_type=jnp.float32)
        + b1_ref[...], 0.0)
    h = jnp.maximum(
        jnp.dot(h, w2_ref[...], preferred_element_type=jnp.float32)
        + b2_ref[...], 0.0)
    out = jnp.sum(h * w3_ref[...], axis=1, keepdims=True) + b3_ref[...]
    out_ref[...] = out


def _finalize(parts, cnts, W1, b1, W2, b2, W3, b3):
    return pl.pallas_call(
        _finalize_body,
        out_shape=jax.ShapeDtypeStruct((NSEG, 1), jnp.float32),
    )(parts, cnts, W1, b1.reshape(1, -1), W2, b2.reshape(1, -1),
      W3.reshape(1, -1), b3.reshape(1, 1))


def kernel(s, segment_ids, W1, b1, W2, b2, W3, b3):
    ids = segment_ids.astype(jnp.int32)
    pad = jnp.full((N_CHUNKS * C - N_NODES,), NSEG, jnp.int32)
    ids2d = jnp.concatenate([ids, pad]).reshape(N_CHUNKS, C)
    parts, cnts = _sc_segment_sums(s, ids2d)
    return _finalize(parts, cnts, W1, b1, W2, b2, W3, b3)


# speedup vs baseline: 6.6289x; 1.3240x over previous
import functools

import jax
import jax.numpy as jnp
from jax import lax
from jax.experimental import pallas as pl
from jax.experimental.pallas import tpu as pltpu
from jax.experimental.pallas import tpu_sc as plsc

N_NODES = 100000
NSEG = 512
D = 128
NC = 2
NS = 16
NW = NC * NS

C = 208
FULL_CHUNKS = N_NODES // C
TAIL = N_NODES - FULL_CHUNKS * C
N_CHUNKS = FULL_CHUNKS + 1
IDS_PAD = N_CHUNKS * C - N_NODES
CROWS = NSEG // 8 + 1


def _sc_segment_sums(s, ids_p):
    mesh = plsc.VectorSubcoreMesh(core_axis_name="c", subcore_axis_name="s")

    @functools.partial(
        pl.kernel,
        mesh=mesh,
        out_type=(
            jax.ShapeDtypeStruct((NW, NSEG, D), jnp.float32),
            jax.ShapeDtypeStruct((NW, CROWS, D), jnp.float32),
        ),
        scratch_types=[
            pltpu.VMEM((NSEG + 1, D), jnp.float32),
            pltpu.VMEM((CROWS, D), jnp.float32),
            pltpu.VMEM((2, C, D), jnp.float32),
            pltpu.VMEM((2, C), jnp.int32),
            pltpu.SemaphoreType.DMA,
            pltpu.SemaphoreType.DMA,
            pltpu.SemaphoreType.DMA,
            pltpu.SemaphoreType.DMA,
        ],
    )
    def seg_sum(s_hbm, ids_hbm, out_parts, cnt_parts, acc, cacc, buf, idxb,
                dsem0, dsem1, isem0, isem1):
        cid = lax.axis_index("c")
        sid = lax.axis_index("s")
        wid = sid * NC + cid

        zeros16 = jnp.zeros((16,), jnp.float32)
        ones16 = jnp.ones((16,), jnp.float32)
        sixteen16 = jnp.full((16,), 16.0, jnp.float32)

        def zero_acc(k, _):
            i = k // (D // 16)
            j = k % (D // 16)
            acc[i, pl.ds(j * 16, 16)] = zeros16
            return 0

        lax.fori_loop(0, (NSEG + 1) * (D // 16), zero_acc, 0)

        def zero_cacc(k, _):
            i = k // (D // 16)
            j = k % (D // 16)
            cacc[i, pl.ds(j * 16, 16)] = zeros16
            return 0

        lax.fori_loop(0, CROWS * (D // 16), zero_cacc, 0)

        def start_in(chunk, slot, dsem, isem):
            @pl.when(chunk < N_CHUNKS)
            def _():
                base = chunk * C
                pltpu.async_copy(ids_hbm.at[chunk], idxb.at[slot], isem)

                @pl.when(chunk < FULL_CHUNKS)
                def _():
                    pltpu.async_copy(
                        s_hbm.at[pl.ds(base, C)], buf.at[slot], dsem)

                @pl.when(chunk == FULL_CHUNKS)
                def _():
                    pltpu.async_copy(
                        s_hbm.at[pl.ds(base, TAIL)],
                        buf.at[slot, pl.ds(0, TAIL)], dsem)

        def wait_in(chunk, slot, dsem, isem):
            base = chunk * C
            pltpu.make_async_copy(
                ids_hbm.at[chunk], idxb.at[slot], isem).wait()

            @pl.when(chunk < FULL_CHUNKS)
            def _():
                pltpu.make_async_copy(
                    s_hbm.at[pl.ds(base, C)], buf.at[slot], dsem).wait()

            @pl.when(chunk == FULL_CHUNKS)
            def _():
                pltpu.make_async_copy(
                    s_hbm.at[pl.ds(base, TAIL)],
                    buf.at[slot, pl.ds(0, TAIL)], dsem).wait()

        def process(chunk, pb):
            def group(g, _):
                rbase = g * 16
                ids16 = idxb[pb, pl.ds(rbase, 16)]
                seg0 = ids16[0]
                seg15 = ids16[15]

                def load_row(r):
                    return [buf[pb, rbase + r, pl.ds(j * 16, 16)]
                            for j in range(D // 16)]

                def store_row(seg, xs):
                    for j in range(D // 16):
                        plsc.addupdate(acc.at[seg, pl.ds(j * 16, 16)], xs[j])
                    plsc.addupdate(
                        cacc.at[seg // 8, pl.ds((seg % 8) * 16, 16)], ones16)

                @pl.when(seg0 == seg15)
                def _():
                    sums = load_row(0)
                    for r in range(1, 16):
                        xs = load_row(r)
                        for j in range(D // 16):
                            sums[j] = sums[j] + xs[j]
                    for j in range(D // 16):
                        plsc.addupdate(
                            acc.at[seg0, pl.ds(j * 16, 16)], sums[j])
                    plsc.addupdate(
                        cacc.at[seg0 // 8, pl.ds((seg0 % 8) * 16, 16)],
                        sixteen16)

                @pl.when(seg0 != seg15)
                def _():
                    xs = load_row(0)
                    seg = seg0
                    for r in range(16):
                        if r < 15:
                            nxt_seg = ids16[r + 1]
                            nxt_xs = []
                            for j in range(D // 16):
                                nxt_xs.append(buf[pb, rbase + r + 1,
                                                  pl.ds(j * 16, 16)])
                                plsc.addupdate(
                                    acc.at[seg, pl.ds(j * 16, 16)], xs[j])
                            plsc.addupdate(
                                cacc.at[seg // 8, pl.ds((seg % 8) * 16, 16)],
                                ones16)
                            xs, seg = nxt_xs, nxt_seg
                        else:
                            store_row(seg, xs)
                return 0

            lax.fori_loop(0, C // 16, group, 0)

        n_my = (N_CHUNKS - 1 - wid) // NW + 1
        start_in(wid, 0, dsem0, isem0)

        def body(t, _):
            chunk = t * NW + wid
            par = lax.rem(t, 2)

            @pl.when(par == 0)
            def _():
                start_in(chunk + NW, 1, dsem1, isem1)
                wait_in(chunk, 0, dsem0, isem0)

            @pl.when(par == 1)
            def _():
                start_in(chunk + NW, 0, dsem0, isem0)
                wait_in(chunk, 1, dsem1, isem1)

            process(chunk, par)
            return 0

        lax.fori_loop(0, n_my, body, 0)

        pltpu.sync_copy(acc.at[pl.ds(0, NSEG)], out_parts.at[wid])
        pltpu.sync_copy(cacc, cnt_parts.at[wid])

    return seg_sum(s, ids_p)


def _finalize_body(parts_ref, cnt_ref, w1_ref, b1_ref, w2_ref, b2_ref,
                   w3_ref, b3_ref, out_ref):
    sums = jnp.sum(parts_ref[...], axis=0)
    cnt = jnp.sum(cnt_ref[...], axis=0)
    srow = lax.broadcasted_iota(jnp.int32, (NSEG, CROWS), 0)
    crow = lax.broadcasted_iota(jnp.int32, (NSEG, CROWS), 1)
    sel = (crow == srow // 8).astype(jnp.float32)
    cntrows = jnp.dot(sel, cnt, preferred_element_type=jnp.float32)
    ss = lax.broadcasted_iota(jnp.int32, (NSEG, D), 0)
    kk = lax.broadcasted_iota(jnp.int32, (NSEG, D), 1)
    lane_mask = (kk == (ss % 8) * 16).astype(jnp.float32)
    counts = jnp.sum(cntrows * lane_mask, axis=1, keepdims=True)
    mean = sums / jnp.maximum(counts, 1.0)
    h = jnp.maximum(
        jnp.dot(mean, w1_ref[...], preferred_element_type=jnp.float32)
        + b1_ref[...], 0.0)
    h = jnp.maximum(
        jnp.dot(h, w2_ref[...], preferred_element_type=jnp.float32)
        + b2_ref[...], 0.0)
    out = jnp.sum(h * w3_ref[...], axis=1, keepdims=True) + b3_ref[...]
    out_ref[...] = out


def _finalize(parts, cnts, W1, b1, W2, b2, W3, b3):
    return pl.pallas_call(
        _finalize_body,
        out_shape=jax.ShapeDtypeStruct((NSEG, 1), jnp.float32),
    )(parts, cnts, W1, b1.reshape(1, -1), W2, b2.reshape(1, -1),
      W3.reshape(1, -1), b3.reshape(1, 1))


def kernel(s, segment_ids, W1, b1, W2, b2, W3, b3):
    ids = segment_ids.astype(jnp.int32)
    pad = jnp.full((IDS_PAD,), NSEG, jnp.int32)
    ids_p = jnp.concatenate([ids, pad]).reshape(N_CHUNKS, C)
    parts, cnts = _sc_segment_sums(s, ids_p)
    return _finalize(parts, cnts, W1, b1, W2, b2, W3, b3)
